# Initial kernel scaffold; baseline (speedup 1.0000x reference)
import jax
import jax.numpy as jnp
from jax.experimental import pallas as pl


def kernel(word, E0, E1, E2):
    raise NotImplementedError("write your pallas kernel here")


# speedup vs baseline: 3.6066x; 3.6066x over previous
import functools

import jax
import jax.numpy as jnp
from jax import lax
from jax.experimental import pallas as pl
from jax.experimental.pallas import tpu as pltpu
from jax.experimental.pallas import tpu_sc as plsc

_D = 50
_DP = 56
_B = 16384
_L = 50
_N = _B * _L
_NC = 2
_NS = 16
_NW = _NC * _NS
_PER_W = _N // _NW
_IW = 128
_IROWS_W = _PER_W // _IW
_CHUNK_IROWS = 8
_CHUNK = _CHUNK_IROWS * _IW
_NCHUNK = _PER_W // _CHUNK


def _body(word_ref, e0, e1, e2, out_ref, idx_v, rows_v, gsem):
    wid = lax.axis_index("s") * _NC + lax.axis_index("c")
    pltpu.sync_copy(word_ref.at[pl.ds(wid * _IROWS_W, _IROWS_W)], idx_v)
    base = wid * _PER_W
    for t, tbl in enumerate((e0, e1, e2)):
        out0 = t * _N + base

        def chunk_body(c, carry, tbl=tbl, out0=out0):
            copies = [
                pltpu.make_async_copy(
                    tbl.at[idx_v.at[c * _CHUNK_IROWS + j]],
                    rows_v.at[pl.ds(j * _IW, _IW)],
                    gsem,
                )
                for j in range(_CHUNK_IROWS)
            ]
            for cp in copies:
                cp.start()
            for cp in copies:
                cp.wait()
            pltpu.sync_copy(rows_v, out_ref.at[pl.ds(out0 + c * _CHUNK, _CHUNK)])
            return carry

        lax.fori_loop(0, _NCHUNK, chunk_body, 0)


_gather = pl.kernel(
    _body,
    mesh=plsc.VectorSubcoreMesh(core_axis_name="c", subcore_axis_name="s"),
    out_type=jax.ShapeDtypeStruct((3 * _N, _DP), jnp.float32),
    compiler_params=pltpu.CompilerParams(use_tc_tiling_on_sc=False),
    scratch_types=[
        pltpu.VMEM((_IROWS_W, _IW), jnp.int32),
        pltpu.VMEM((_CHUNK, _DP), jnp.float32),
        pltpu.SemaphoreType.DMA,
    ],
)


@jax.jit
def _run(word2d, E0, E1, E2):
    return _gather(word2d, E0, E1, E2)[:, : _D].reshape(3 * _B, _L, _D)


def kernel(word, E0, E1, E2):
    word2d = word.reshape(-1).astype(jnp.int32).reshape(_N // _IW, _IW)
    pad = ((0, 0), (0, _DP - _D))
    return _run(word2d, jnp.pad(E0, pad), jnp.pad(E1, pad), jnp.pad(E2, pad))


# speedup vs baseline: 5.4925x; 1.5229x over previous
import functools

import jax
import jax.numpy as jnp
from jax import lax
from jax.experimental import pallas as pl
from jax.experimental.pallas import tpu as pltpu
from jax.experimental.pallas import tpu_sc as plsc

_V = 100000
_VP = 100096
_D = 50
_DP = 56
_B = 16384
_L = 50
_LP = 56
_OUTB = 3 * _B
_NC = 2
_NS = 16
_CB = 8192
_NG = _CB // 128


def _body(wordT, e0, e1, e2, out, col_v, idx_v, stage_v):
    cid = lax.axis_index("c")
    sid = lax.axis_index("s")
    wid = sid * _NC + cid

    def process(tbl, t, d):
        pltpu.sync_copy(tbl.at[d, pl.ds(0, _V)], col_v)

        def l_body(l, carry):
            def h_body(h, carry2):
                pltpu.sync_copy(wordT.at[l, pl.ds(h * _CB, _CB)], idx_v)

                def g_body(i, c3):
                    base = i * 128
                    for j in range(8):
                        off = pl.multiple_of(base + j * 16, 16)
                        vals = plsc.load_gather(col_v, [idx_v[pl.ds(off, 16)]])
                        stage_v[pl.ds(off, 16)] = vals
                    return c3

                lax.fori_loop(0, _NG, g_body, 0)
                pltpu.sync_copy(
                    stage_v, out.at[l, d, pl.ds(t * _B + h * _CB, _CB)])
                return carry2

            lax.fori_loop(0, 2, h_body, 0)
            return carry

        lax.fori_loop(0, _L, l_body, 0)

    for t, tbl in enumerate((e0, e1, e2)):
        process(tbl, t, wid)
        a = (0, 18, 4)[t]
        r = lax.rem(wid - a + 32, 32)

        @pl.when(r < 18)
        def _(tbl=tbl, t=t, r=r):
            process(tbl, t, 32 + r)


_col_gather = pl.kernel(
    _body,
    mesh=plsc.VectorSubcoreMesh(core_axis_name="c", subcore_axis_name="s"),
    out_type=jax.ShapeDtypeStruct((_L, _DP, _OUTB), jnp.float32),
    compiler_params=pltpu.CompilerParams(
        use_tc_tiling_on_sc=False, needs_layout_passes=False),
    scratch_types=[
        pltpu.VMEM((_V,), jnp.float32),
        pltpu.VMEM((_CB,), jnp.int32),
        pltpu.VMEM((_CB,), jnp.float32),
    ],
)


@jax.jit
def _run(wordT, t0, t1, t2):
    outp = _col_gather(wordT, t0, t1, t2)
    return outp.transpose(2, 0, 1)[:, :, :_D]


def kernel(word, E0, E1, E2):
    wordT = jnp.pad(word.astype(jnp.int32).T, ((0, _LP - _L), (0, 0)))
    tp = lambda E: jnp.pad(E.T, ((0, _DP - _D), (0, _VP - _V)))
    return _run(wordT, tp(E0), tp(E1), tp(E2))


# speedup vs baseline: 7.6208x; 1.3875x over previous
import functools

import jax
import jax.numpy as jnp
from jax import lax
from jax.experimental import pallas as pl
from jax.experimental.pallas import tpu as pltpu
from jax.experimental.pallas import tpu_sc as plsc

_V = 100000
_VP = 100096
_D = 50
_DP = 56
_B = 16384
_L = 50
_LP = 56
_OUTB = 3 * _B
_NC = 2
_NS = 16
_CB = 4096
_NH = _B // _CB
_NG = _CB // 128
_NSTEP = _L * _NH


def _body(wordT, e0, e1, e2, out,
          col_v, idx_a, idx_b, stage_a, stage_b,
          sem_ia, sem_ib, sem_oa, sem_ob):
    cid = lax.axis_index("c")
    sid = lax.axis_index("s")
    wid = sid * _NC + cid

    def idx_copy(q, buf, sem):
        l = q // _NH
        h = lax.rem(q, _NH)
        return pltpu.make_async_copy(
            wordT.at[l, pl.ds(h * _CB, _CB)], buf, sem)

    def out_copy(q, t, d, buf, sem):
        l = q // _NH
        h = lax.rem(q, _NH)
        return pltpu.make_async_copy(
            buf, out.at[l, d, pl.ds(t * _B + h * _CB, _CB)], sem)

    def gather(idx_v, stage_v):
        def g_body(i, c3):
            base = i * 128
            for j in range(8):
                off = pl.multiple_of(base + j * 16, 16)
                vals = plsc.load_gather(col_v, [idx_v[pl.ds(off, 16)]])
                stage_v[pl.ds(off, 16)] = vals
            return c3

        lax.fori_loop(0, _NG, g_body, 0)

    def process(tbl, t, d):
        pltpu.sync_copy(tbl.at[d, pl.ds(0, _V)], col_v)
        idx_copy(0, idx_a, sem_ia).start()

        def step(q, buf_i, buf_s, sem_i, sem_o, sem_i_next, buf_i_next):
            idx_copy(q, buf_i, sem_i).wait()

            @pl.when(q + 1 < _NSTEP)
            def _():
                idx_copy(q + 1, buf_i_next, sem_i_next).start()

            @pl.when(q >= 2)
            def _():
                out_copy(q - 2, t, d, buf_s, sem_o).wait()

            gather(buf_i, buf_s)
            out_copy(q, t, d, buf_s, sem_o).start()

        def pair(k, carry):
            step(2 * k, idx_a, stage_a, sem_ia, sem_oa, sem_ib, idx_b)
            step(2 * k + 1, idx_b, stage_b, sem_ib, sem_ob, sem_ia, idx_a)
            return carry

        lax.fori_loop(0, _NSTEP // 2, pair, 0)
        out_copy(_NSTEP - 2, t, d, stage_a, sem_oa).wait()
        out_copy(_NSTEP - 1, t, d, stage_b, sem_ob).wait()

    for t, tbl in enumerate((e0, e1, e2)):
        process(tbl, t, wid)
        a = (0, 18, 4)[t]
        r = lax.rem(wid - a + 32, 32)

        @pl.when(r < 18)
        def _(tbl=tbl, t=t, r=r):
            process(tbl, t, 32 + r)


_col_gather = pl.kernel(
    _body,
    mesh=plsc.VectorSubcoreMesh(core_axis_name="c", subcore_axis_name="s"),
    out_type=jax.ShapeDtypeStruct((_L, _DP, _OUTB), jnp.float32),
    compiler_params=pltpu.CompilerParams(
        use_tc_tiling_on_sc=False, needs_layout_passes=False),
    scratch_types=[
        pltpu.VMEM((_V,), jnp.float32),
        pltpu.VMEM((_CB,), jnp.int32),
        pltpu.VMEM((_CB,), jnp.int32),
        pltpu.VMEM((_CB,), jnp.float32),
        pltpu.VMEM((_CB,), jnp.float32),
        pltpu.SemaphoreType.DMA,
        pltpu.SemaphoreType.DMA,
        pltpu.SemaphoreType.DMA,
        pltpu.SemaphoreType.DMA,
    ],
)


@jax.jit
def _run(wordT, t0, t1, t2):
    outp = _col_gather(wordT, t0, t1, t2)
    return outp.transpose(2, 0, 1)[:, :, :_D]


def kernel(word, E0, E1, E2):
    wordT = jnp.pad(word.astype(jnp.int32).T, ((0, _LP - _L), (0, 0)))
    tp = lambda E: jnp.pad(E.T, ((0, _DP - _D), (0, _VP - _V)))
    return _run(wordT, tp(E0), tp(E1), tp(E2))


# speedup vs baseline: 12.1879x; 1.5993x over previous
import functools

import jax
import jax.numpy as jnp
from jax import lax
from jax.experimental import pallas as pl
from jax.experimental.pallas import tpu as pltpu
from jax.experimental.pallas import tpu_sc as plsc

_V = 100000
_VP = 100096
_D = 50
_DP = 56
_B = 16384
_L = 50
_LP = 56
_OUTB = 3 * _B
_NC = 2
_NS = 16
_CB = 4096
_NH = _B // _CB
_NG = _CB // 128
_NSTEP = _L * _NH


def _body(wordT, e0, e1, e2, out,
          col_v, idx_a, idx_b, stage_a, stage_b,
          sem_ia, sem_ib, sem_oa, sem_ob):
    cid = lax.axis_index("c")
    sid = lax.axis_index("s")
    wid = sid * _NC + cid

    def idx_copy(q, buf, sem):
        l = q // _NH
        h = lax.rem(q, _NH)
        return pltpu.make_async_copy(
            wordT.at[l, pl.ds(h * _CB, _CB)], buf, sem)

    def out_copy(q, t, dblk, di, buf, sem):
        l = q // _NH
        h = lax.rem(q, _NH)
        bblk0 = (t * _B + h * _CB) // 128
        return pltpu.make_async_copy(
            buf, out.at[l, dblk, pl.ds(bblk0, _CB // 128), di, :], sem)

    def gather(idx_v, stage_v):
        @plsc.parallel_loop(0, _NG, unroll=4)
        def g_body(i):
            for j in range(8):
                off = pl.multiple_of(j * 16, 16)
                vals = plsc.load_gather(col_v, [idx_v[pl.ds(i * 128 + off, 16)]])
                stage_v[i, pl.ds(off, 16)] = vals

    def process(tbl, t, d):
        dblk = d // 8
        di = lax.rem(d, 8)
        pltpu.sync_copy(tbl.at[d, pl.ds(0, _V)], col_v)
        idx_copy(0, idx_a, sem_ia).start()

        def step(q, buf_i, buf_s, sem_i, sem_o, sem_i_next, buf_i_next):
            idx_copy(q, buf_i, sem_i).wait()

            @pl.when(q + 1 < _NSTEP)
            def _():
                idx_copy(q + 1, buf_i_next, sem_i_next).start()

            @pl.when(q >= 2)
            def _():
                out_copy(q - 2, t, dblk, di, buf_s, sem_o).wait()

            gather(buf_i, buf_s)
            out_copy(q, t, dblk, di, buf_s, sem_o).start()

        def pair(k, carry):
            step(2 * k, idx_a, stage_a, sem_ia, sem_oa, sem_ib, idx_b)
            step(2 * k + 1, idx_b, stage_b, sem_ib, sem_ob, sem_ia, idx_a)
            return carry

        lax.fori_loop(0, _NSTEP // 2, pair, 0)
        out_copy(_NSTEP - 2, t, dblk, di, stage_a, sem_oa).wait()
        out_copy(_NSTEP - 1, t, dblk, di, stage_b, sem_ob).wait()

    for t, tbl in enumerate((e0, e1, e2)):
        process(tbl, t, wid)
        a = (0, 18, 4)[t]
        r = lax.rem(wid - a + 32, 32)

        @pl.when(r < 18)
        def _(tbl=tbl, t=t, r=r):
            process(tbl, t, 32 + r)


_col_gather = pl.kernel(
    _body,
    mesh=plsc.VectorSubcoreMesh(core_axis_name="c", subcore_axis_name="s"),
    out_type=jax.ShapeDtypeStruct((_L, _DP // 8, _OUTB // 128, 8, 128),
                                  jnp.float32),
    compiler_params=pltpu.CompilerParams(
        use_tc_tiling_on_sc=False, needs_layout_passes=False),
    scratch_types=[
        pltpu.VMEM((_V,), jnp.float32),
        pltpu.VMEM((_CB,), jnp.int32),
        pltpu.VMEM((_CB,), jnp.int32),
        pltpu.VMEM((_CB // 128, 128), jnp.float32),
        pltpu.VMEM((_CB // 128, 128), jnp.float32),
        pltpu.SemaphoreType.DMA,
        pltpu.SemaphoreType.DMA,
        pltpu.SemaphoreType.DMA,
        pltpu.SemaphoreType.DMA,
    ],
)


@jax.jit
def _run(wordT, t0, t1, t2):
    outp = _col_gather(wordT, t0, t1, t2)
    out = outp.transpose(2, 4, 0, 1, 3).reshape(_OUTB, _L, _DP)
    return out[:, :, :_D]


def kernel(word, E0, E1, E2):
    wordT = jnp.pad(word.astype(jnp.int32).T, ((0, _LP - _L), (0, 0)))
    tp = lambda E: jnp.pad(E.T, ((0, _DP - _D), (0, _VP - _V)))
    return _run(wordT, tp(E0), tp(E1), tp(E2))


# speedup vs baseline: 18.7433x; 1.5379x over previous
import functools

import jax
import jax.numpy as jnp
from jax import lax
from jax.experimental import pallas as pl
from jax.experimental.pallas import tpu as pltpu
from jax.experimental.pallas import tpu_sc as plsc

_V = 100000
_VP = 100096
_D = 50
_DP = 56
_B = 16384
_L = 50
_LP = 56
_OUTB = 3 * _B
_NC = 2
_NS = 16
_CB = 4096
_NH = _B // _CB
_NG = _CB // 128
_NSTEP = _L * _NH


def _body(wordT, e0, e1, e2, out,
          col_v, idx_a, idx_b, stage_a, stage_b,
          sem_ia, sem_ib, sem_oa, sem_ob):
    cid = lax.axis_index("c")
    sid = lax.axis_index("s")
    wid = sid * _NC + cid

    def idx_copy(l, half, buf, sem):
        return pltpu.make_async_copy(
            wordT.at[l, pl.ds(half * (2 * _CB), 2 * _CB)], buf, sem)

    def out_copy(q, t, dblk, di, buf, sem):
        l = q // _NH
        h = lax.rem(q, _NH)
        bblk0 = (t * _B + h * _CB) // 128
        return pltpu.make_async_copy(
            buf, out.at[l, dblk, pl.ds(bblk0, _CB // 128), di, :], sem)

    def gather(idx_v, base, stage_v):
        @plsc.parallel_loop(0, _NG, unroll=4)
        def g_body(i):
            for j in range(8):
                off = pl.multiple_of(j * 16, 16)
                vals = plsc.load_gather(
                    col_v, [idx_v[pl.ds(base + i * 128 + off, 16)]])
                stage_v[i, pl.ds(off, 16)] = vals

    def process(tbl, t, d):
        dblk = d // 8
        di = lax.rem(d, 8)
        pltpu.sync_copy(tbl.at[d, pl.ds(0, _V)], col_v)
        idx_copy(0, 0, idx_a, sem_ia).start()
        idx_copy(0, 1, idx_b, sem_ib).start()

        def step(q, buf_i, base, buf_s, sem_o):
            @pl.when(q >= 2)
            def _():
                out_copy(q - 2, t, dblk, di, buf_s, sem_o).wait()

            gather(buf_i, base, buf_s)
            out_copy(q, t, dblk, di, buf_s, sem_o).start()

        def l_iter(m, carry):
            q0 = _NH * m
            idx_copy(m, 0, idx_a, sem_ia).wait()
            step(q0, idx_a, 0, stage_a, sem_oa)
            step(q0 + 1, idx_a, _CB, stage_b, sem_ob)

            @pl.when(m + 1 < _L)
            def _():
                idx_copy(m + 1, 0, idx_a, sem_ia).start()

            idx_copy(m, 1, idx_b, sem_ib).wait()
            step(q0 + 2, idx_b, 0, stage_a, sem_oa)
            step(q0 + 3, idx_b, _CB, stage_b, sem_ob)

            @pl.when(m + 1 < _L)
            def _():
                idx_copy(m + 1, 1, idx_b, sem_ib).start()

            return carry

        lax.fori_loop(0, _L, l_iter, 0)
        out_copy(_NSTEP - 2, t, dblk, di, stage_a, sem_oa).wait()
        out_copy(_NSTEP - 1, t, dblk, di, stage_b, sem_ob).wait()

    for t, tbl in enumerate((e0, e1, e2)):
        process(tbl, t, wid)
        a = (0, 18, 4)[t]
        r = lax.rem(wid - a + 32, 32)

        @pl.when(r < 18)
        def _(tbl=tbl, t=t, r=r):
            process(tbl, t, 32 + r)


_col_gather = pl.kernel(
    _body,
    mesh=plsc.VectorSubcoreMesh(core_axis_name="c", subcore_axis_name="s"),
    out_type=jax.ShapeDtypeStruct((_L, _DP // 8, _OUTB // 128, 8, 128),
                                  jnp.float32),
    compiler_params=pltpu.CompilerParams(
        use_tc_tiling_on_sc=False, needs_layout_passes=False),
    scratch_types=[
        pltpu.VMEM((_V,), jnp.float32),
        pltpu.VMEM((2 * _CB,), jnp.int32),
        pltpu.VMEM((2 * _CB,), jnp.int32),
        pltpu.VMEM((_CB // 128, 128), jnp.float32),
        pltpu.VMEM((_CB // 128, 128), jnp.float32),
        pltpu.SemaphoreType.DMA,
        pltpu.SemaphoreType.DMA,
        pltpu.SemaphoreType.DMA,
        pltpu.SemaphoreType.DMA,
    ],
)


@jax.jit
def _run(wordT, t0, t1, t2):
    outp = _col_gather(wordT, t0, t1, t2)
    out = outp.transpose(2, 4, 0, 1, 3).reshape(_OUTB, _L, _DP)
    return out[:, :, :_D]


def kernel(word, E0, E1, E2):
    wordT = jnp.pad(word.astype(jnp.int32).T, ((0, _LP - _L), (0, 0)))
    tp = lambda E: jnp.pad(E.T, ((0, _DP - _D), (0, _VP - _V)))
    return _run(wordT, tp(E0), tp(E1), tp(E2))
